# Initial kernel scaffold; baseline (speedup 1.0000x reference)
import functools

import jax
import jax.numpy as jnp
from jax import lax
from jax.experimental import pallas as pl
from jax.experimental.pallas import tpu as pltpu
from jax.experimental.pallas import tpu_sc as plsc

BATCH = 16384
INPUT_DIM = 39
N_CAT = 26
VOCAB = 100000
EMB = 16
N_CONT = INPUT_DIM - N_CAT
OUT_D = N_CAT * EMB + N_CONT

NC, NS = 2, 16
NW = NC * NS
CB = BATCH // NW
SUB = 128
NSUB = CB // SUB


def _body(w_hbm, idx_hbm, cont_hbm, out_hbm, idx_v, rows_v, cont_v, sem):
    wid = lax.axis_index("s") * NC + lax.axis_index("c")
    base = wid * CB

    pltpu.sync_copy(idx_hbm.at[wid], idx_v)

    pltpu.sync_copy(cont_hbm.at[pl.ds(base, CB)], cont_v)
    pltpu.sync_copy(cont_v, out_hbm.at[pl.ds(base, CB), pl.ds(N_CAT * EMB, N_CONT)])

    def feat(f, carry):
        descs = [
            pltpu.async_copy(
                w_hbm.at[idx_v.at[f, c]], rows_v.at[pl.ds(c * SUB, SUB)], sem
            )
            for c in range(NSUB)
        ]
        for d in descs:
            d.wait()
        pltpu.sync_copy(rows_v, out_hbm.at[pl.ds(base, CB), pl.ds(f * EMB, EMB)])
        return carry

    lax.fori_loop(0, N_CAT, feat, 0)


_emb_kernel = functools.partial(
    pl.kernel,
    out_type=jax.ShapeDtypeStruct((BATCH, OUT_D), jnp.float32),
    mesh=plsc.VectorSubcoreMesh(core_axis_name="c", subcore_axis_name="s"),
    scratch_types=[
        pltpu.VMEM((N_CAT, NSUB, SUB), jnp.int32),
        pltpu.VMEM((CB, EMB), jnp.float32),
        pltpu.VMEM((CB, N_CONT), jnp.float32),
        pltpu.SemaphoreType.DMA,
    ],
)(_body)


def kernel(x, W):
    offs = (jnp.arange(N_CAT, dtype=jnp.int32) * VOCAB)[:, None]
    xoff = x[:, :N_CAT].T + offs
    idx = xoff.reshape(N_CAT, NW, NSUB, SUB).transpose(1, 0, 2, 3)
    cont = x[:, N_CAT:].astype(jnp.float32)
    w_flat = W.reshape(N_CAT * VOCAB, EMB)
    return _emb_kernel(w_flat, idx, cont)


# speedup vs baseline: 1.2341x; 1.2341x over previous
import functools

import jax
import jax.numpy as jnp
from jax import lax
from jax.experimental import pallas as pl
from jax.experimental.pallas import tpu as pltpu
from jax.experimental.pallas import tpu_sc as plsc

BATCH = 16384
INPUT_DIM = 39
N_CAT = 26
VOCAB = 100000
EMB = 16
N_CONT = INPUT_DIM - N_CAT
OUT_D = N_CAT * EMB + N_CONT

NC, NS = 2, 16
NW = NC * NS
CB = BATCH // NW
SUB = 128
NSUB = CB // SUB


def _body(w_hbm, idx_hbm, cont_hbm, out_hbm, idx_v, rows_v, cont_v, sem):
    wid = lax.axis_index("s") * NC + lax.axis_index("c")
    base = wid * CB

    pltpu.sync_copy(idx_hbm.at[wid], idx_v)

    pltpu.sync_copy(cont_hbm.at[pl.ds(base, CB)], cont_v)
    pltpu.sync_copy(cont_v, out_hbm.at[pl.ds(base, CB), pl.ds(N_CAT * EMB, N_CONT)])

    def feat(f, carry):
        descs = [
            pltpu.async_copy(
                w_hbm.at[idx_v.at[f, c]], rows_v.at[pl.ds(c * SUB, SUB)], sem
            )
            for c in range(NSUB)
        ]
        for d in descs:
            d.wait()
        pltpu.sync_copy(rows_v, out_hbm.at[pl.ds(base, CB), pl.ds(f * EMB, EMB)])
        return carry

    lax.fori_loop(0, N_CAT, feat, 0)


_emb_kernel = functools.partial(
    pl.kernel,
    out_type=jax.ShapeDtypeStruct((BATCH, OUT_D), jnp.float32),
    mesh=plsc.VectorSubcoreMesh(core_axis_name="c", subcore_axis_name="s"),
    compiler_params=pltpu.CompilerParams(use_tc_tiling_on_sc=False),
    scratch_types=[
        pltpu.VMEM((N_CAT, NSUB, SUB), jnp.int32),
        pltpu.VMEM((CB, EMB), jnp.float32),
        pltpu.VMEM((CB, N_CONT), jnp.float32),
        pltpu.SemaphoreType.DMA,
    ],
)(_body)


def kernel(x, W):
    offs = (jnp.arange(N_CAT, dtype=jnp.int32) * VOCAB)[:, None]
    xoff = x[:, :N_CAT].T + offs
    idx = xoff.reshape(N_CAT, NW, NSUB, SUB).transpose(1, 0, 2, 3)
    cont = x[:, N_CAT:].astype(jnp.float32)
    w_flat = W.reshape(N_CAT * VOCAB, EMB)
    return _emb_kernel(w_flat, idx, cont)
